# baseline (speedup 1.0000x reference)
import functools

import jax
import jax.numpy as jnp
from jax import lax
from jax.experimental import pallas as pl
from jax.experimental.pallas import tpu as pltpu
from jax.experimental.pallas import tpu_sc as plsc

N_VARS = 10000
N_CLAUSES = 42000
W = 3
F = 128
Q = 32
ROUNDS = 8
EPS = 1e-6

VP = 10000
CP = 42000
BV = 400
BC = 200
NEG_SLOPE = 0.2


def _leaky(x):
    return jnp.where(x >= 0, x, NEG_SLOPE * x)


def _dot(x, w):
    return jnp.dot(x, w, preferred_element_type=jnp.float32)


_row = lambda i: (i, 0)
_full = lambda i: (0, 0)


def _wspecs(arrs):
    return [pl.BlockSpec(a.shape, _full) for a in arrs]


def _qtail_body(h1, w2, b2, w3, b3, out):
    h = _leaky(_dot(h1[...], w2[...]) + b2[...])
    out[...] = _dot(h, w3[...]) + b3[...]


def _query_tail(h1, params):
    (w2, b2), (w3, b3) = params[1], params[2]
    args = (w2, b2[None], w3, b3[None])
    return pl.pallas_call(
        _qtail_body,
        grid=(VP // BV,),
        in_specs=[pl.BlockSpec((BV, h1.shape[1]), _row)] + _wspecs(args),
        out_specs=pl.BlockSpec((BV, Q), _row),
        out_shape=jax.ShapeDtypeStruct((VP, Q), jnp.float32),
    )(h1, *args)


def _clause_body(cs, cv, w1, b1, w2, b2, w3, b3, vl, csp):
    cu = jnp.concatenate([cs[...], cv[...]], axis=1)
    h = _leaky(_dot(cu, w1[...]) + b1[...])
    h = _leaky(_dot(h, w2[...]) + b2[...])
    cd = _dot(h, w3[...]) + b3[...]
    vl[...] = cd[:, :Q]
    csp[...] = cd[:, Q:]


def _clause_stage(cs, cv, params):
    (w1, b1), (w2, b2), (w3, b3) = params
    args = (w1, b1[None], w2, b2[None], w3, b3[None])
    return pl.pallas_call(
        _clause_body,
        grid=(CP // BC,),
        in_specs=[pl.BlockSpec((BC, F), _row), pl.BlockSpec((BC, Q), _row)]
        + _wspecs(args),
        out_specs=[pl.BlockSpec((BC, Q), _row), pl.BlockSpec((BC, F), _row)],
        out_shape=[jax.ShapeDtypeStruct((CP, Q), jnp.float32),
                   jax.ShapeDtypeStruct((CP, F), jnp.float32)],
    )(cs, cv, *args)


def _cfin_body(csp, cs_old, mean, scale, cs_new):
    t = (csp[...] - mean[...]) * scale[0, 0]
    cs_new[...] = t * 0.25 + 0.1 * cs_old[...]


def _clause_finalize(csp, cs_old, mean, scale):
    return pl.pallas_call(
        _cfin_body,
        grid=(CP // BC,),
        in_specs=[pl.BlockSpec((BC, F), _row), pl.BlockSpec((BC, F), _row),
                  pl.BlockSpec((1, F), _full), pl.BlockSpec((1, 1), _full)],
        out_specs=pl.BlockSpec((BC, F), _row),
        out_shape=jax.ShapeDtypeStruct((CP, F), jnp.float32),
    )(csp, cs_old, mean, scale)


def _utail_body(h1, w2, b2, w3, b3, out):
    h = _leaky(_dot(h1[...], w2[...]) + b2[...])
    out[...] = _dot(h, w3[...]) + b3[...]


def _update_tail(h1, params):
    (w2, b2), (w3, b3) = params[1], params[2]
    args = (w2, b2[None], w3, b3[None])
    return pl.pallas_call(
        _utail_body,
        grid=(VP // BV,),
        in_specs=[pl.BlockSpec((BV, h1.shape[1]), _row)] + _wspecs(args),
        out_specs=pl.BlockSpec((BV, F), _row),
        out_shape=jax.ShapeDtypeStruct((VP, F), jnp.float32),
    )(h1, *args)


def _vfin_body(x, v_old, mean, scale, w1, b1, w2, b2, w3, b3, v_new, logits):
    t = (x[...] - mean[...]) * scale[0, 0]
    vn = t * 0.25 + 0.1 * v_old[...]
    v_new[...] = vn
    h = _leaky(_dot(vn, w1[...]) + b1[...])
    h = _leaky(_dot(h, w2[...]) + b2[...])
    lo = _dot(h, w3[...]) + b3[...]
    logits[...] = lo + jnp.zeros((BV, 16), jnp.float32)


def _var_finalize(x, v_old, mean, scale, params):
    (w1, b1), (w2, b2), (w3, b3) = params
    args = (w1, b1[None], w2, b2[None], w3, b3[None])
    return pl.pallas_call(
        _vfin_body,
        grid=(VP // BV,),
        in_specs=[pl.BlockSpec((BV, F), _row), pl.BlockSpec((BV, F), _row),
                  pl.BlockSpec((1, F), _full), pl.BlockSpec((1, 1), _full)]
        + _wspecs(args),
        out_specs=[pl.BlockSpec((BV, F), _row), pl.BlockSpec((BV, 16), _row)],
        out_shape=[jax.ShapeDtypeStruct((VP, F), jnp.float32),
                   jax.ShapeDtypeStruct((VP, 16), jnp.float32)],
    )(x, v_old, mean, scale, *args)


CLP = 42240
CROWS = CLP // 128
LROWS = CROWS


def _loss_body(ll0, ll1, ll2, sg0, sg1, sg2, loss_acc, unsat_acc):
    r = jax.lax.broadcasted_iota(jnp.int32, (LROWS, 128), 0)
    c = jax.lax.broadcasted_iota(jnp.int32, (LROWS, 128), 1)
    valid = (r * 128 + c) < N_CLAUSES
    a0, a1, a2 = ll0[...], ll1[...], ll2[...]
    s0, s1, s2 = sg0[...], sg1[...], sg2[...]
    cv = (jax.nn.sigmoid(-a0 * s0) * jax.nn.sigmoid(-a1 * s1)
          * jax.nn.sigmoid(-a2 * s2))
    pcl = cv * (-jnp.log(1.0 - cv + EPS))
    pcl = jnp.where(valid, pcl, 0.0)
    b0 = jnp.round(jax.nn.sigmoid(a0))
    b1_ = jnp.round(jax.nn.sigmoid(a1))
    b2_ = jnp.round(jax.nn.sigmoid(a2))
    w0 = (s0 + 1.0) * 0.5
    w1_ = (s1 + 1.0) * 0.5
    w2_ = (s2 + 1.0) * 0.5
    sat = jnp.maximum(jnp.maximum((b0 == w0).astype(jnp.float32),
                                  (b1_ == w1_).astype(jnp.float32)),
                      (b2_ == w2_).astype(jnp.float32))
    unsat = jnp.where(valid, 1.0 - sat, 0.0)
    loss_acc[...] = jnp.sum(pcl, axis=0, keepdims=True)
    unsat_acc[...] = jnp.sum(unsat, axis=0, keepdims=True)


def _loss_stage(ll0, ll1, ll2, sg0, sg1, sg2):
    return pl.pallas_call(
        _loss_body,
        grid=(1,),
        in_specs=[pl.BlockSpec((LROWS, 128), _row)] * 6,
        out_specs=[pl.BlockSpec((1, 128), _full)] * 2,
        out_shape=[jax.ShapeDtypeStruct((1, 128), jnp.float32)] * 2,
    )(ll0, ll1, ll2, sg0, sg1, sg2)


NW = 32
LB = 126976
B_PER_W = LB // NW
NCHUNK = B_PER_W // 128


def _sc_gather128(table, idx):
    mesh = plsc.VectorSubcoreMesh(core_axis_name="c", subcore_axis_name="s")

    def body(table_hbm, idx_hbm, out_hbm, idx_v, rows_a, rows_b, sem, wsem):
        wid = lax.axis_index("s") * 2 + lax.axis_index("c")
        base = wid * B_PER_W
        pltpu.sync_copy(idx_hbm.at[pl.ds(base, B_PER_W)], idx_v)
        bufs = (rows_a, rows_b)
        stages = []
        done = 0
        while done < NCHUNK:
            n = min(2, NCHUNK - done)
            stages.append((done, n))
            done += n
        gath = [None, None]
        wout = [None, None]
        for s, (off, n) in enumerate(stages):
            buf = bufs[s % 2]
            if wout[s % 2] is not None:
                wout[s % 2].wait()
                wout[s % 2] = None
            gath[s % 2] = [
                pltpu.async_copy(
                    table_hbm.at[idx_v.at[pl.ds((off + j) * 128, 128)]],
                    buf.at[pl.ds(j * 128, 128), :], sem)
                for j in range(n)
            ]
            if s >= 1:
                ps, (poff, pn) = s - 1, stages[s - 1]
                for c in gath[ps % 2]:
                    c.wait()
                gath[ps % 2] = None
                wout[ps % 2] = pltpu.async_copy(
                    bufs[ps % 2].at[pl.ds(0, pn * 128), :],
                    out_hbm.at[pl.ds(base + poff * 128, pn * 128)], wsem)
        ls, (loff, lnn) = len(stages) - 1, stages[-1]
        for c in gath[ls % 2]:
            c.wait()
        last = pltpu.async_copy(bufs[ls % 2].at[pl.ds(0, lnn * 128), :],
                                out_hbm.at[pl.ds(base + loff * 128, lnn * 128)], wsem)
        for h in wout:
            if h is not None:
                h.wait()
        last.wait()

    kern = functools.partial(
        pl.kernel, mesh=mesh,
        out_type=jax.ShapeDtypeStruct((LB, 128), jnp.float32),
        scratch_types=[
            pltpu.VMEM((B_PER_W,), jnp.int32),
            pltpu.VMEM((256, 128), jnp.float32),
            pltpu.VMEM((256, 128), jnp.float32),
            pltpu.SemaphoreType.DMA,
            pltpu.SemaphoreType.DMA,
        ])(body)
    return kern(table, idx)


def _sel_body(g, m, out, *, parts):
    acc = None
    span = 128 // parts
    for k in range(parts):
        t = g[:, k * span:(k + 1) * span] * m[:, k:k + 1]
        acc = t if acc is None else acc + t
    out[...] = acc


def _lane_select(g128, masks, parts):
    width = 128 // parts
    body = functools.partial(_sel_body, parts=parts)
    return pl.pallas_call(
        body,
        grid=(LB // 256,),
        in_specs=[pl.BlockSpec((256, 128), _row), pl.BlockSpec((256, parts), _row)],
        out_specs=pl.BlockSpec((256, width), _row),
        out_shape=jax.ShapeDtypeStruct((LB, width), jnp.float32),
    )(g128, masks)


def _zero_state_padded(n_pad, n_valid, f):
    col0 = jnp.zeros((n_pad, 1), jnp.float32) + (1.0 - 1.0 / f)
    rest = jnp.full((n_pad, f - 1), -1.0 / f, jnp.float32)
    x = jnp.concatenate([col0, rest], axis=1) * (jnp.sqrt(float(f)) * 0.25)
    rows = jnp.arange(n_pad)[:, None]
    return jnp.where(rows < n_valid, x, 0.0)


def _pair_stats(x, mask, counts):
    mean = jax.ops.segment_sum(x, mask, num_segments=1) / counts
    xc = x - mean[0:1]
    sq = jnp.sum(jnp.square(xc), axis=-1, keepdims=True)
    gnorm = jax.ops.segment_sum(sq, mask, num_segments=1) / counts
    scale = jax.lax.rsqrt(gnorm + EPS)
    return mean, scale


def kernel(clauses, variable_count, clauses_count, params):
    key = jax.random.key(42)
    num_graphs = variable_count.shape[0]
    graph_id = jnp.arange(num_graphs)
    variables_mask = jnp.repeat(graph_id, variable_count, total_repeat_length=N_VARS)
    clauses_mask = jnp.repeat(graph_id, clauses_count, total_repeat_length=N_CLAUSES)
    vcounts = jnp.maximum(jax.ops.segment_sum(
        jnp.ones((N_VARS,), jnp.float32), variables_mask, num_segments=1), 1.0)[:, None]
    ccounts = jnp.maximum(jax.ops.segment_sum(
        jnp.ones((N_CLAUSES,), jnp.float32), clauses_mask, num_segments=1), 1.0)[:, None]

    flat_lits = clauses.reshape(-1)
    clause_ids = jnp.repeat(jnp.arange(N_CLAUSES), W)
    var_idx = jnp.abs(flat_lits) - 1
    signs = jnp.sign(flat_lits).astype(jnp.float32)[:, None]
    pos_mask = (flat_lits > 0).astype(jnp.float32)[:, None]
    nlit = W * N_CLAUSES
    idx_q = jnp.pad(var_idx // 4, (0, LB - nlit))
    idx_l = jnp.pad(var_idx // 8, (0, LB - nlit))
    mq = jnp.pad(jax.nn.one_hot(var_idx % 4, 4, dtype=jnp.float32),
                 ((0, LB - nlit), (0, 0)))
    ml = jnp.pad(jax.nn.one_hot(var_idx % 8, 8, dtype=jnp.float32),
                 ((0, LB - nlit), (0, 0)))
    sg2 = jnp.sign(clauses).astype(jnp.float32)
    sgl = [jnp.pad(sg2[:, j], (0, CLP - N_CLAUSES)).reshape(CROWS, 128)
           for j in range(W)]

    (qw1, qb1) = params['variables_query'][0]
    (uw1, ub1) = params['update_gate'][0]

    variables = _zero_state_padded(VP, N_VARS, F)
    clause_state = _zero_state_padded(CP, N_CLAUSES, F)
    loss_acc = jnp.zeros((), jnp.float32)
    last_logits = jnp.zeros((N_VARS, 1), jnp.float32)
    active = jnp.array(True)

    for step in range(ROUNDS):
        noise = jax.random.normal(jax.random.fold_in(key, step), (N_VARS, 4), jnp.float32)
        v1 = jnp.concatenate([variables, noise], axis=-1)
        h1q = jax.nn.leaky_relu(v1 @ qw1 + qb1, negative_slope=NEG_SLOPE)
        query = _query_tail(h1q, params['variables_query'])
        g128 = _sc_gather128(query.reshape(VP * Q // 128, 128), idx_q)
        lit_vals = _lane_select(g128, mq, 4)[:nlit] * signs
        sp = jax.nn.softplus(lit_vals)
        csum = jax.ops.segment_sum(sp, clause_ids, num_segments=N_CLAUSES)
        cv = jnp.exp(-csum)
        ncv3 = jnp.broadcast_to((-cv)[:, None, :], (N_CLAUSES, W, Q)).reshape(W * N_CLAUSES, Q)
        ct_lit = ncv3 * jnp.exp(lit_vals - sp)
        grad = jnp.zeros((N_VARS, Q), jnp.float32).at[var_idx].add(ct_lit * signs)
        vl, csp = _clause_stage(clause_state, cv, params['clause_mlp'])
        cmean, cscale = _pair_stats(csp, clauses_mask, ccounts)
        cs_new = _clause_finalize(csp, clause_state, cmean, cscale)
        lpl = jnp.broadcast_to(vl[:, None, :],
                               (N_CLAUSES, W, Q)).reshape(W * N_CLAUSES, Q)
        ln64 = jnp.zeros((N_VARS, 2 * Q), jnp.float32).at[var_idx].add(
            jnp.concatenate([lpl * pos_mask, lpl * (1.0 - pos_mask)], axis=1))
        vlp = ln64[:, :Q]
        vln = ln64[:, Q:]
        unit = jnp.concatenate([variables, grad, vlp, vln], axis=-1)
        h1u = jax.nn.leaky_relu(unit @ uw1 + ub1, negative_slope=NEG_SLOPE)
        x = _update_tail(h1u, params['update_gate'])
        vmean, vscale = _pair_stats(x, variables_mask, vcounts)
        v_new, logits8 = _var_finalize(x, variables, vmean, vscale,
                                       params['variables_output'])
        lflat = logits8[:, 0]
        gl = _sc_gather128(logits8.reshape(VP * 16 // 128, 128), idx_l)
        ll3 = _lane_select(gl, ml, 8)[:nlit, 0].reshape(N_CLAUSES, W)
        lls = [jnp.pad(ll3[:, j], (0, CLP - N_CLAUSES)).reshape(CROWS, 128)
               for j in range(W)]
        lsum, usat = _loss_stage(lls[0], lls[1], lls[2], sgl[0], sgl[1], sgl[2])
        logit_loss = jnp.sqrt(jnp.sum(lsum) + 1e-6)
        n_unsat = jnp.sum(usat)
        loss_acc = loss_acc + jnp.where(active, logit_loss, 0.0)
        last_logits = jnp.where(active, lflat[:, None], last_logits)
        variables = jnp.where(active, v_new, variables)
        clause_state = jnp.where(active, cs_new, clause_state)
        active = jnp.logical_and(active, n_unsat > 0.5)

    return last_logits, loss_acc / float(ROUNDS)


# speedup vs baseline: 1.0617x; 1.0617x over previous
import functools

import jax
import jax.numpy as jnp
from jax import lax
from jax.experimental import pallas as pl
from jax.experimental.pallas import tpu as pltpu
from jax.experimental.pallas import tpu_sc as plsc

N_VARS = 10000
N_CLAUSES = 42000
W = 3
F = 128
Q = 32
ROUNDS = 8
EPS = 1e-6

VP = 10000
CP = 42000
BV = 400
BC = 200
NEG_SLOPE = 0.2


def _leaky(x):
    return jnp.where(x >= 0, x, NEG_SLOPE * x)


def _dot(x, w):
    return jnp.dot(x, w, preferred_element_type=jnp.float32)


_row = lambda i: (i, 0)
_full = lambda i: (0, 0)


def _wspecs(arrs):
    return [pl.BlockSpec(a.shape, _full) for a in arrs]


def _qtail_body(h1, w2, b2, w3, b3, out):
    h = _leaky(_dot(h1[...], w2[...]) + b2[...])
    out[...] = _dot(h, w3[...]) + b3[...]


def _query_tail(h1, params):
    (w2, b2), (w3, b3) = params[1], params[2]
    args = (w2, b2[None], w3, b3[None])
    return pl.pallas_call(
        _qtail_body,
        grid=(VP // BV,),
        in_specs=[pl.BlockSpec((BV, h1.shape[1]), _row)] + _wspecs(args),
        out_specs=pl.BlockSpec((BV, Q), _row),
        out_shape=jax.ShapeDtypeStruct((VP, Q), jnp.float32),
    )(h1, *args)


def _clause_body(cs, cv, w1, b1, w2, b2, w3, b3, vl, csp):
    cu = jnp.concatenate([cs[...], cv[...]], axis=1)
    h = _leaky(_dot(cu, w1[...]) + b1[...])
    h = _leaky(_dot(h, w2[...]) + b2[...])
    cd = _dot(h, w3[...]) + b3[...]
    vl[...] = cd[:, :Q]
    csp[...] = cd[:, Q:]


def _clause_stage(cs, cv, params):
    (w1, b1), (w2, b2), (w3, b3) = params
    args = (w1, b1[None], w2, b2[None], w3, b3[None])
    return pl.pallas_call(
        _clause_body,
        grid=(CP // BC,),
        in_specs=[pl.BlockSpec((BC, F), _row), pl.BlockSpec((BC, Q), _row)]
        + _wspecs(args),
        out_specs=[pl.BlockSpec((BC, Q), _row), pl.BlockSpec((BC, F), _row)],
        out_shape=[jax.ShapeDtypeStruct((CP, Q), jnp.float32),
                   jax.ShapeDtypeStruct((CP, F), jnp.float32)],
    )(cs, cv, *args)


def _cfin_body(csp, cs_old, mean, scale, cs_new):
    t = (csp[...] - mean[...]) * scale[0, 0]
    cs_new[...] = t * 0.25 + 0.1 * cs_old[...]


def _clause_finalize(csp, cs_old, mean, scale):
    return pl.pallas_call(
        _cfin_body,
        grid=(CP // BC,),
        in_specs=[pl.BlockSpec((BC, F), _row), pl.BlockSpec((BC, F), _row),
                  pl.BlockSpec((1, F), _full), pl.BlockSpec((1, 1), _full)],
        out_specs=pl.BlockSpec((BC, F), _row),
        out_shape=jax.ShapeDtypeStruct((CP, F), jnp.float32),
    )(csp, cs_old, mean, scale)


def _utail_body(h1, w2, b2, w3, b3, out):
    h = _leaky(_dot(h1[...], w2[...]) + b2[...])
    out[...] = _dot(h, w3[...]) + b3[...]


def _update_tail(h1, params):
    (w2, b2), (w3, b3) = params[1], params[2]
    args = (w2, b2[None], w3, b3[None])
    return pl.pallas_call(
        _utail_body,
        grid=(VP // BV,),
        in_specs=[pl.BlockSpec((BV, h1.shape[1]), _row)] + _wspecs(args),
        out_specs=pl.BlockSpec((BV, F), _row),
        out_shape=jax.ShapeDtypeStruct((VP, F), jnp.float32),
    )(h1, *args)


def _vfin_body(x, v_old, mean, scale, w1, b1, w2, b2, w3, b3, v_new, logits):
    t = (x[...] - mean[...]) * scale[0, 0]
    vn = t * 0.25 + 0.1 * v_old[...]
    v_new[...] = vn
    h = _leaky(_dot(vn, w1[...]) + b1[...])
    h = _leaky(_dot(h, w2[...]) + b2[...])
    lo = _dot(h, w3[...]) + b3[...]
    logits[...] = lo + jnp.zeros((BV, 16), jnp.float32)


def _var_finalize(x, v_old, mean, scale, params):
    (w1, b1), (w2, b2), (w3, b3) = params
    args = (w1, b1[None], w2, b2[None], w3, b3[None])
    return pl.pallas_call(
        _vfin_body,
        grid=(VP // BV,),
        in_specs=[pl.BlockSpec((BV, F), _row), pl.BlockSpec((BV, F), _row),
                  pl.BlockSpec((1, F), _full), pl.BlockSpec((1, 1), _full)]
        + _wspecs(args),
        out_specs=[pl.BlockSpec((BV, F), _row), pl.BlockSpec((BV, 16), _row)],
        out_shape=[jax.ShapeDtypeStruct((VP, F), jnp.float32),
                   jax.ShapeDtypeStruct((VP, 16), jnp.float32)],
    )(x, v_old, mean, scale, *args)


CLP = 42240
CROWS = CLP // 128
LROWS = CROWS


def _loss_body(ll0, ll1, ll2, sg0, sg1, sg2, loss_acc, unsat_acc):
    r = jax.lax.broadcasted_iota(jnp.int32, (LROWS, 128), 0)
    c = jax.lax.broadcasted_iota(jnp.int32, (LROWS, 128), 1)
    valid = (r * 128 + c) < N_CLAUSES
    a0, a1, a2 = ll0[...], ll1[...], ll2[...]
    s0, s1, s2 = sg0[...], sg1[...], sg2[...]
    cv = (jax.nn.sigmoid(-a0 * s0) * jax.nn.sigmoid(-a1 * s1)
          * jax.nn.sigmoid(-a2 * s2))
    pcl = cv * (-jnp.log(1.0 - cv + EPS))
    pcl = jnp.where(valid, pcl, 0.0)
    b0 = jnp.round(jax.nn.sigmoid(a0))
    b1_ = jnp.round(jax.nn.sigmoid(a1))
    b2_ = jnp.round(jax.nn.sigmoid(a2))
    w0 = (s0 + 1.0) * 0.5
    w1_ = (s1 + 1.0) * 0.5
    w2_ = (s2 + 1.0) * 0.5
    sat = jnp.maximum(jnp.maximum((b0 == w0).astype(jnp.float32),
                                  (b1_ == w1_).astype(jnp.float32)),
                      (b2_ == w2_).astype(jnp.float32))
    unsat = jnp.where(valid, 1.0 - sat, 0.0)
    loss_acc[...] = jnp.sum(pcl, axis=0, keepdims=True)
    unsat_acc[...] = jnp.sum(unsat, axis=0, keepdims=True)


def _loss_stage(ll0, ll1, ll2, sg0, sg1, sg2):
    return pl.pallas_call(
        _loss_body,
        grid=(1,),
        in_specs=[pl.BlockSpec((LROWS, 128), _row)] * 6,
        out_specs=[pl.BlockSpec((1, 128), _full)] * 2,
        out_shape=[jax.ShapeDtypeStruct((1, 128), jnp.float32)] * 2,
    )(ll0, ll1, ll2, sg0, sg1, sg2)


NW = 32
LB = 126976
B_PER_W = LB // NW
NCHUNK = B_PER_W // 128


def _sc_gather128(table, idx):
    mesh = plsc.VectorSubcoreMesh(core_axis_name="c", subcore_axis_name="s")

    def body(table_hbm, idx_hbm, out_hbm, idx_v, rows_a, rows_b, sem, wsem):
        wid = lax.axis_index("s") * 2 + lax.axis_index("c")
        base = wid * B_PER_W
        pltpu.sync_copy(idx_hbm.at[pl.ds(base, B_PER_W)], idx_v)
        bufs = (rows_a, rows_b)
        stages = []
        done = 0
        while done < NCHUNK:
            n = min(2, NCHUNK - done)
            stages.append((done, n))
            done += n
        gath = [None, None]
        wout = [None, None]
        for s, (off, n) in enumerate(stages):
            buf = bufs[s % 2]
            if wout[s % 2] is not None:
                wout[s % 2].wait()
                wout[s % 2] = None
            gath[s % 2] = [
                pltpu.async_copy(
                    table_hbm.at[idx_v.at[pl.ds((off + j) * 128, 128)]],
                    buf.at[pl.ds(j * 128, 128), :], sem)
                for j in range(n)
            ]
            if s >= 1:
                ps, (poff, pn) = s - 1, stages[s - 1]
                for c in gath[ps % 2]:
                    c.wait()
                gath[ps % 2] = None
                wout[ps % 2] = pltpu.async_copy(
                    bufs[ps % 2].at[pl.ds(0, pn * 128), :],
                    out_hbm.at[pl.ds(base + poff * 128, pn * 128)], wsem)
        ls, (loff, lnn) = len(stages) - 1, stages[-1]
        for c in gath[ls % 2]:
            c.wait()
        last = pltpu.async_copy(bufs[ls % 2].at[pl.ds(0, lnn * 128), :],
                                out_hbm.at[pl.ds(base + loff * 128, lnn * 128)], wsem)
        for h in wout:
            if h is not None:
                h.wait()
        last.wait()

    kern = functools.partial(
        pl.kernel, mesh=mesh,
        out_type=jax.ShapeDtypeStruct((LB, 128), jnp.float32),
        scratch_types=[
            pltpu.VMEM((B_PER_W,), jnp.int32),
            pltpu.VMEM((256, 128), jnp.float32),
            pltpu.VMEM((256, 128), jnp.float32),
            pltpu.SemaphoreType.DMA,
            pltpu.SemaphoreType.DMA,
        ])(body)
    return kern(table, idx)


def _sel_body(g, m, out, *, parts):
    acc = None
    span = 128 // parts
    for k in range(parts):
        t = g[:, k * span:(k + 1) * span] * m[:, k:k + 1]
        acc = t if acc is None else acc + t
    out[...] = acc


def _lane_select(g128, masks, parts):
    width = 128 // parts
    body = functools.partial(_sel_body, parts=parts)
    return pl.pallas_call(
        body,
        grid=(LB // 256,),
        in_specs=[pl.BlockSpec((256, 128), _row), pl.BlockSpec((256, parts), _row)],
        out_specs=pl.BlockSpec((256, width), _row),
        out_shape=jax.ShapeDtypeStruct((LB, width), jnp.float32),
    )(g128, masks)


def _zero_state_padded(n_pad, n_valid, f):
    col0 = jnp.zeros((n_pad, 1), jnp.float32) + (1.0 - 1.0 / f)
    rest = jnp.full((n_pad, f - 1), -1.0 / f, jnp.float32)
    x = jnp.concatenate([col0, rest], axis=1) * (jnp.sqrt(float(f)) * 0.25)
    rows = jnp.arange(n_pad)[:, None]
    return jnp.where(rows < n_valid, x, 0.0)


def _pair_stats(x, mask, counts):
    mean = jax.ops.segment_sum(x, mask, num_segments=1) / counts
    xc = x - mean[0:1]
    sq = jnp.sum(jnp.square(xc), axis=-1, keepdims=True)
    gnorm = jax.ops.segment_sum(sq, mask, num_segments=1) / counts
    scale = jax.lax.rsqrt(gnorm + EPS)
    return mean, scale


def kernel(clauses, variable_count, clauses_count, params):
    key = jax.random.key(42)
    num_graphs = variable_count.shape[0]
    graph_id = jnp.arange(num_graphs)
    variables_mask = jnp.repeat(graph_id, variable_count, total_repeat_length=N_VARS)
    clauses_mask = jnp.repeat(graph_id, clauses_count, total_repeat_length=N_CLAUSES)
    vcounts = jnp.maximum(jax.ops.segment_sum(
        jnp.ones((N_VARS,), jnp.float32), variables_mask, num_segments=1), 1.0)[:, None]
    ccounts = jnp.maximum(jax.ops.segment_sum(
        jnp.ones((N_CLAUSES,), jnp.float32), clauses_mask, num_segments=1), 1.0)[:, None]

    flat_lits = clauses.reshape(-1)
    clause_ids = jnp.repeat(jnp.arange(N_CLAUSES), W)
    var_idx = jnp.abs(flat_lits) - 1
    signs = jnp.sign(flat_lits).astype(jnp.float32)[:, None]
    pos_mask = (flat_lits > 0).astype(jnp.float32)[:, None]
    nlit = W * N_CLAUSES
    idx_q = jnp.pad(var_idx // 4, (0, LB - nlit))
    idx_l = jnp.pad(var_idx // 8, (0, LB - nlit))
    mq = jnp.pad(jax.nn.one_hot(var_idx % 4, 4, dtype=jnp.float32),
                 ((0, LB - nlit), (0, 0)))
    ml = jnp.pad(jax.nn.one_hot(var_idx % 8, 8, dtype=jnp.float32),
                 ((0, LB - nlit), (0, 0)))
    sg2 = jnp.sign(clauses).astype(jnp.float32)
    sgl = [jnp.pad(sg2[:, j], (0, CLP - N_CLAUSES)).reshape(CROWS, 128)
           for j in range(W)]

    (qw1, qb1) = params['variables_query'][0]
    (uw1, ub1) = params['update_gate'][0]

    variables = _zero_state_padded(VP, N_VARS, F)
    clause_state = _zero_state_padded(CP, N_CLAUSES, F)
    loss_acc = jnp.zeros((), jnp.float32)
    last_logits = jnp.zeros((N_VARS, 1), jnp.float32)
    active = jnp.array(True)

    for step in range(ROUNDS):
        noise = jax.random.normal(jax.random.fold_in(key, step), (N_VARS, 4), jnp.float32)
        v1 = jnp.concatenate([variables, noise], axis=-1)
        h1q = jax.nn.leaky_relu(v1 @ qw1 + qb1, negative_slope=NEG_SLOPE)
        query = _query_tail(h1q, params['variables_query'])
        lit_vals = query[var_idx] * signs
        sp = jax.nn.softplus(lit_vals)
        csum = jax.ops.segment_sum(sp, clause_ids, num_segments=N_CLAUSES)
        cv = jnp.exp(-csum)
        ncv3 = jnp.broadcast_to((-cv)[:, None, :], (N_CLAUSES, W, Q)).reshape(W * N_CLAUSES, Q)
        ct_lit = ncv3 * jnp.exp(lit_vals - sp)
        grad = jnp.zeros((N_VARS, Q), jnp.float32).at[var_idx].add(ct_lit * signs)
        vl, csp = _clause_stage(clause_state, cv, params['clause_mlp'])
        cmean, cscale = _pair_stats(csp, clauses_mask, ccounts)
        cs_new = _clause_finalize(csp, clause_state, cmean, cscale)
        lpl = jnp.broadcast_to(vl[:, None, :],
                               (N_CLAUSES, W, Q)).reshape(W * N_CLAUSES, Q)
        ln64 = jnp.zeros((N_VARS, 2 * Q), jnp.float32).at[var_idx].add(
            jnp.concatenate([lpl * pos_mask, lpl * (1.0 - pos_mask)], axis=1))
        vlp = ln64[:, :Q]
        vln = ln64[:, Q:]
        unit = jnp.concatenate([variables, grad, vlp, vln], axis=-1)
        h1u = jax.nn.leaky_relu(unit @ uw1 + ub1, negative_slope=NEG_SLOPE)
        x = _update_tail(h1u, params['update_gate'])
        vmean, vscale = _pair_stats(x, variables_mask, vcounts)
        v_new, logits8 = _var_finalize(x, variables, vmean, vscale,
                                       params['variables_output'])
        lflat = logits8[:, 0]
        gl = _sc_gather128(logits8.reshape(VP * 16 // 128, 128), idx_l)
        ll3 = _lane_select(gl, ml, 8)[:nlit, 0].reshape(N_CLAUSES, W)
        lls = [jnp.pad(ll3[:, j], (0, CLP - N_CLAUSES)).reshape(CROWS, 128)
               for j in range(W)]
        lsum, usat = _loss_stage(lls[0], lls[1], lls[2], sgl[0], sgl[1], sgl[2])
        logit_loss = jnp.sqrt(jnp.sum(lsum) + 1e-6)
        n_unsat = jnp.sum(usat)
        loss_acc = loss_acc + jnp.where(active, logit_loss, 0.0)
        last_logits = jnp.where(active, lflat[:, None], last_logits)
        variables = jnp.where(active, v_new, variables)
        clause_state = jnp.where(active, cs_new, clause_state)
        active = jnp.logical_and(active, n_unsat > 0.5)

    return last_logits, loss_acc / float(ROUNDS)
